# Initial kernel scaffold; baseline (speedup 1.0000x reference)
import jax
import jax.numpy as jnp
from jax.experimental import pallas as pl


def kernel(x, edge_index, Ws1, Wn1, b1, Ws2, Wn2, b2):
    raise NotImplementedError("write your pallas kernel here")


# speedup vs baseline: 3.6911x; 3.6911x over previous
import functools

import jax
import jax.numpy as jnp
from jax import lax
from jax.experimental import pallas as pl
from jax.experimental.pallas import tpu as pltpu
from jax.experimental.pallas import tpu_sc as plsc

N_NODES = 10000
N_EDGES = 320000
D = 128
W_AUG = 144

NC, NS = 2, 16
NW = NC * NS
E_PER_W = N_EDGES // NW
CHUNK = 80
N_CHUNKS = E_PER_W // CHUNK
N_PAD = 10240
ROWS_PER_SUB = N_PAD // NS


def _make_sc_agg(width):
  mesh = plsc.VectorSubcoreMesh(core_axis_name="c", subcore_axis_name="s")

  @functools.partial(
      pl.kernel,
      out_type=jax.ShapeDtypeStruct((NC, N_PAD, width), jnp.float32),
      mesh=mesh,
      scratch_types=[
          pltpu.VMEM((CHUNK,), jnp.int32),
          pltpu.VMEM((CHUNK,), jnp.int32),
          pltpu.VMEM((CHUNK, width), jnp.float32),
          pltpu.VMEM_SHARED((N_PAD, width), jnp.float32),
          pltpu.SemaphoreType.DMA,
      ],
      compiler_params=pltpu.CompilerParams(use_tc_tiling_on_sc=False),
  )
  def agg(z_hbm, src_hbm, dst_hbm, zeros_hbm, out_hbm,
          src_v, dst_v, rows_v, acc, sem):
    cid = lax.axis_index("c")
    sid = lax.axis_index("s")
    wid = sid * NC + cid
    r0 = sid * ROWS_PER_SUB
    pltpu.sync_copy(zeros_hbm, acc.at[pl.ds(r0, ROWS_PER_SUB)])
    plsc.subcore_barrier()

    base = wid * E_PER_W

    def body(i, carry):
      off = base + i * CHUNK
      pltpu.sync_copy(src_hbm.at[pl.ds(off, CHUNK)], src_v)
      pltpu.sync_copy(dst_hbm.at[pl.ds(off, CHUNK)], dst_v)
      pltpu.async_copy(z_hbm.at[src_v], rows_v, sem).wait()
      pltpu.sync_copy(rows_v, acc.at[dst_v], add=True)
      return carry

    lax.fori_loop(0, N_CHUNKS, body, 0)
    plsc.subcore_barrier()
    pltpu.sync_copy(acc.at[pl.ds(r0, ROWS_PER_SUB)],
                    out_hbm.at[cid, pl.ds(r0, ROWS_PER_SUB)])

  return agg


_sc_agg_aug = _make_sc_agg(W_AUG)
_sc_agg_plain = _make_sc_agg(D)


BR = 1000


def _tc_layer1_body(x_ref, a0_ref, a1_ref, d0_ref, d1_ref,
                    ws_ref, wn_ref, b_ref, h_ref, inv_ref):
  deg = d0_ref[:, 0] + d1_ref[:, 0]
  inv = 1.0 / jnp.maximum(deg, 1.0)
  mean = (a0_ref[...] + a1_ref[...]) * inv[:, None]
  h = (jnp.dot(x_ref[...], ws_ref[...], preferred_element_type=jnp.float32)
       + jnp.dot(mean, wn_ref[...], preferred_element_type=jnp.float32)
       + b_ref[...])
  h_ref[...] = jnp.maximum(h, 0.0)
  inv_ref[...] = jnp.broadcast_to(inv[:, None], (BR, 16))


def _tc_layer2_body(h_ref, a0_ref, a1_ref, inv_ref,
                    ws_ref, wn_ref, b_ref, out_ref):
  inv = inv_ref[:, 0]
  mean = (a0_ref[...] + a1_ref[...]) * inv[:, None]
  out_ref[...] = (
      jnp.dot(h_ref[...], ws_ref[...], preferred_element_type=jnp.float32)
      + jnp.dot(mean, wn_ref[...], preferred_element_type=jnp.float32)
      + b_ref[...])


def _row_spec(w):
  return pl.BlockSpec((BR, w), lambda i: (i, 0))


def _full_spec(shape):
  return pl.BlockSpec(shape, lambda i: tuple(0 for _ in shape))


_tc_layer1 = pl.pallas_call(
    _tc_layer1_body,
    grid=(N_NODES // BR,),
    in_specs=[
        _row_spec(D), _row_spec(D), _row_spec(D),
        _row_spec(16), _row_spec(16),
        _full_spec((D, D)), _full_spec((D, D)), _full_spec((1, D)),
    ],
    out_specs=[_row_spec(D), _row_spec(16)],
    out_shape=[
        jax.ShapeDtypeStruct((N_NODES, D), jnp.float32),
        jax.ShapeDtypeStruct((N_NODES, 16), jnp.float32),
    ],
)

_tc_layer2 = pl.pallas_call(
    _tc_layer2_body,
    grid=(N_NODES // BR,),
    in_specs=[
        _row_spec(D), _row_spec(D), _row_spec(D), _row_spec(16),
        _full_spec((D, D)), _full_spec((D, D)), _full_spec((1, D)),
    ],
    out_specs=_row_spec(D),
    out_shape=jax.ShapeDtypeStruct((N_NODES, D), jnp.float32),
)


@jax.jit
def kernel(x, edge_index, Ws1, Wn1, b1, Ws2, Wn2, b2):
  src = edge_index[0].astype(jnp.int32)
  dst = edge_index[1].astype(jnp.int32)
  x = x.astype(jnp.float32)
  ones_col = jnp.ones((N_NODES, 1), jnp.float32)
  pad = jnp.zeros((N_NODES, W_AUG - D - 1), jnp.float32)
  xaug = jnp.concatenate([x, ones_col, pad], axis=1)
  zeros_aug = jnp.zeros((ROWS_PER_SUB, W_AUG), jnp.float32)
  zeros_d = jnp.zeros((ROWS_PER_SUB, D), jnp.float32)

  agg1 = _sc_agg_aug(xaug, src, dst, zeros_aug)
  a0 = agg1[0, :N_NODES, :D]
  a1 = agg1[1, :N_NODES, :D]
  d0 = agg1[0, :N_NODES, D:D + 16]
  d1 = agg1[1, :N_NODES, D:D + 16]
  h, inv = _tc_layer1(x, a0, a1, d0, d1, Ws1, Wn1, b1.reshape(1, D))

  agg2 = _sc_agg_plain(h, src, dst, zeros_d)
  out = _tc_layer2(h, agg2[0, :N_NODES], agg2[1, :N_NODES], inv,
                   Ws2, Wn2, b2.reshape(1, D))
  return out


# speedup vs baseline: 9.4118x; 2.5499x over previous
import functools

import jax
import jax.numpy as jnp
from jax import lax
from jax.experimental import pallas as pl
from jax.experimental.pallas import tpu as pltpu
from jax.experimental.pallas import tpu_sc as plsc

N_NODES = 10000
N_EDGES = 320000
D = 128

NC, NS = 2, 16
NW = NC * NS
E_PER_W = N_EDGES // NW
CHUNK = 80
N_CHUNKS = E_PER_W // CHUNK
N_PAD = 10240
ROWS_PER_SUB = N_PAD // NS


def _make_sc_agg(with_deg):
  mesh = plsc.VectorSubcoreMesh(core_axis_name="c", subcore_axis_name="s")

  out_type = [jax.ShapeDtypeStruct((NC, N_PAD, D), jnp.float32)]
  scratch = [
      pltpu.VMEM((N_CHUNKS, CHUNK), jnp.int32),
      pltpu.VMEM((N_CHUNKS, CHUNK), jnp.int32),
      pltpu.VMEM((CHUNK, D), jnp.float32),
      pltpu.VMEM((CHUNK, D), jnp.float32),
      pltpu.VMEM_SHARED((N_PAD, D), jnp.float32),
      pltpu.SemaphoreType.DMA,
      pltpu.SemaphoreType.DMA,
  ]
  if with_deg:
    out_type.append(jax.ShapeDtypeStruct((NC, N_PAD, 8), jnp.float32))
    scratch += [
        pltpu.VMEM((CHUNK, 8), jnp.float32),
        pltpu.VMEM_SHARED((N_PAD, 8), jnp.float32),
    ]

  @functools.partial(
      pl.kernel,
      out_type=out_type,
      mesh=mesh,
      scratch_types=scratch,
      compiler_params=pltpu.CompilerParams(use_tc_tiling_on_sc=False),
  )
  def agg(z_hbm, src_hbm, dst_hbm, zeros_hbm, *rest):
    if with_deg:
      ones_hbm, out_hbm, deg_hbm, src_v, dst_v, rows_a, rows_b, acc, \
          sem_a, sem_b, ones_v, dacc = rest
    else:
      out_hbm, src_v, dst_v, rows_a, rows_b, acc, sem_a, sem_b = rest
    cid = lax.axis_index("c")
    sid = lax.axis_index("s")
    wid = sid * NC + cid
    r0 = sid * ROWS_PER_SUB
    pltpu.sync_copy(src_hbm.at[wid], src_v)
    pltpu.sync_copy(dst_hbm.at[wid], dst_v)
    pltpu.sync_copy(zeros_hbm.at[pl.ds(0, ROWS_PER_SUB)],
                    acc.at[pl.ds(r0, ROWS_PER_SUB)])
    if with_deg:
      pltpu.sync_copy(ones_hbm, ones_v)
      pltpu.sync_copy(zeros_hbm.at[pl.ds(0, ROWS_PER_SUB), pl.ds(0, 8)],
                      dacc.at[pl.ds(r0, ROWS_PER_SUB)])
    plsc.subcore_barrier()

    def scatter(rows, g):
      pltpu.sync_copy(rows, acc.at[dst_v.at[g]], add=True)
      if with_deg:
        pltpu.sync_copy(ones_v, dacc.at[dst_v.at[g]], add=True)

    gather0 = pltpu.async_copy(z_hbm.at[src_v.at[0]], rows_a, sem_a)

    def body(i, carry):
      g = 2 * i
      gather0.wait()
      desc_b = pltpu.async_copy(z_hbm.at[src_v.at[g + 1]], rows_b, sem_b)
      scatter(rows_a, g)
      desc_b.wait()
      pltpu.async_copy(z_hbm.at[src_v.at[g + 2]], rows_a, sem_a)
      scatter(rows_b, g + 1)
      return carry

    lax.fori_loop(0, (N_CHUNKS - 1) // 2, body, 0)
    gather0.wait()
    scatter(rows_a, N_CHUNKS - 1)
    plsc.subcore_barrier()
    pltpu.sync_copy(acc.at[pl.ds(r0, ROWS_PER_SUB)],
                    out_hbm.at[cid, pl.ds(r0, ROWS_PER_SUB)])
    if with_deg:
      pltpu.sync_copy(dacc.at[pl.ds(r0, ROWS_PER_SUB)],
                      deg_hbm.at[cid, pl.ds(r0, ROWS_PER_SUB)])

  return agg


_sc_agg_deg = _make_sc_agg(True)
_sc_agg_plain = _make_sc_agg(False)


BR = 1000


def _tc_layer1_body(x_ref, a0_ref, a1_ref, d0_ref, d1_ref,
                    ws_ref, wn_ref, b_ref, h_ref, inv_ref):
  deg = d0_ref[:, 0] + d1_ref[:, 0]
  inv = 1.0 / jnp.maximum(deg, 1.0)
  mean = (a0_ref[...] + a1_ref[...]) * inv[:, None]
  h = (jnp.dot(x_ref[...], ws_ref[...], preferred_element_type=jnp.float32)
       + jnp.dot(mean, wn_ref[...], preferred_element_type=jnp.float32)
       + b_ref[...])
  h_ref[...] = jnp.maximum(h, 0.0)
  inv_ref[...] = jnp.broadcast_to(inv[:, None], (BR, 16))


def _tc_layer2_body(h_ref, a0_ref, a1_ref, inv_ref,
                    ws_ref, wn_ref, b_ref, out_ref):
  inv = inv_ref[:, 0]
  mean = (a0_ref[...] + a1_ref[...]) * inv[:, None]
  out_ref[...] = (
      jnp.dot(h_ref[...], ws_ref[...], preferred_element_type=jnp.float32)
      + jnp.dot(mean, wn_ref[...], preferred_element_type=jnp.float32)
      + b_ref[...])


def _row_spec(w):
  return pl.BlockSpec((BR, w), lambda i: (i, 0))


def _full_spec(shape):
  return pl.BlockSpec(shape, lambda i: tuple(0 for _ in shape))


_tc_layer1 = pl.pallas_call(
    _tc_layer1_body,
    grid=(N_NODES // BR,),
    in_specs=[
        _row_spec(D), _row_spec(D), _row_spec(D),
        _row_spec(8), _row_spec(8),
        _full_spec((D, D)), _full_spec((D, D)), _full_spec((1, D)),
    ],
    out_specs=[_row_spec(D), _row_spec(16)],
    out_shape=[
        jax.ShapeDtypeStruct((N_NODES, D), jnp.float32),
        jax.ShapeDtypeStruct((N_NODES, 16), jnp.float32),
    ],
)

_tc_layer2 = pl.pallas_call(
    _tc_layer2_body,
    grid=(N_NODES // BR,),
    in_specs=[
        _row_spec(D), _row_spec(D), _row_spec(D), _row_spec(16),
        _full_spec((D, D)), _full_spec((D, D)), _full_spec((1, D)),
    ],
    out_specs=_row_spec(D),
    out_shape=jax.ShapeDtypeStruct((N_NODES, D), jnp.float32),
)


@jax.jit
def kernel(x, edge_index, Ws1, Wn1, b1, Ws2, Wn2, b2):
  src = edge_index[0].astype(jnp.int32).reshape(NW, N_CHUNKS, CHUNK)
  dst = edge_index[1].astype(jnp.int32).reshape(NW, N_CHUNKS, CHUNK)
  x = x.astype(jnp.float32)
  zeros_d = jnp.zeros((ROWS_PER_SUB, D), jnp.float32)
  ones_8 = jnp.ones((CHUNK, 8), jnp.float32)

  agg1, deg = _sc_agg_deg(x, src, dst, zeros_d, ones_8)
  d0 = deg[0, :N_NODES]
  d1 = deg[1, :N_NODES]
  h, inv = _tc_layer1(x, agg1[0, :N_NODES], agg1[1, :N_NODES], d0, d1,
                      Ws1, Wn1, b1.reshape(1, D))

  agg2 = _sc_agg_plain(h, src, dst, zeros_d)[0]
  out = _tc_layer2(h, agg2[0, :N_NODES], agg2[1, :N_NODES], inv,
                   Ws2, Wn2, b2.reshape(1, D))
  return out
